# baseline (speedup 1.0000x reference)
import jax
import jax.numpy as jnp
from jax.experimental import pallas as pl
from jax.experimental.pallas import tpu as pltpu

K = 3
ALPHA = 8.3

_CT = 128
_HC = 16
_G = 4


def _shift_h(x, o):
    if o == 0:
        return x
    z = jnp.zeros_like(x[..., :1, :])
    if o == 1:
        return jnp.concatenate([x[..., 1:, :], z], axis=-2)
    return jnp.concatenate([z, x[..., :-1, :]], axis=-2)


def _shift_w(x, o):
    if o == 0:
        return x
    z = jnp.zeros_like(x[..., :, :1])
    if o == 1:
        return jnp.concatenate([x[..., :, 1:], z], axis=-1)
    return jnp.concatenate([z, x[..., :, :-1]], axis=-1)


def _weights_body(depth_ref, wn_ref):
    d = depth_ref[0, 0]
    ws = []
    for oi in (-1, 0, 1):
        dh = _shift_h(d, oi)
        for oj in (-1, 0, 1):
            dk = _shift_w(dh, oj)
            ws.append(jnp.exp(-ALPHA * jnp.abs(d - dk)))
    den = ws[0]
    for w in ws[1:]:
        den = den + w
    inv = 1.0 / den
    k = 0
    for oi in (-1, 0, 1):
        for oj in (-1, 0, 1):
            wn_ref[0, k] = _shift_w(ws[k] * inv, -oj)
            k += 1


def _pool_body(wn_ref, img_ref, out_ref):
    H = out_ref.shape[2]
    zrow = jnp.zeros((1, out_ref.shape[3]), jnp.float32)

    def _xh(c, h0):
        xh = {}
        for oi in (-1, 0, 1):
            s = h0 + oi
            if s < 0:
                xh[oi] = jnp.concatenate(
                    [zrow, img_ref[0, c, 0:_HC - 1, :]], axis=0)
            elif s + _HC > H:
                xh[oi] = jnp.concatenate(
                    [img_ref[0, c, s:H, :], zrow], axis=0)
            else:
                xh[oi] = img_ref[0, c, s:s + _HC, :]
        return xh

    for c0 in range(0, _CT, _G):
        for h0 in range(0, H, _HC):
            xhs = [_xh(c0 + g, h0) for g in range(_G)]
            yss = [[None] * K for _ in range(_G)]
            for j_idx in range(K):
                for i_idx, di in enumerate((-1, 0, 1)):
                    w = wn_ref[0, i_idx * K + j_idx, h0:h0 + _HC, :]
                    for g in range(_G):
                        t = w * xhs[g][di]
                        y = yss[g][j_idx]
                        yss[g][j_idx] = t if y is None else y + t
            for g in range(_G):
                ys = yss[g]
                acc = (pltpu.roll(ys[0], 1, axis=1) + ys[1]
                       + pltpu.roll(ys[2], out_ref.shape[3] - 1, axis=1))
                out_ref[0, c0 + g, h0:h0 + _HC, :] = acc


def kernel(img, depth):
    B, C, H, W = img.shape
    wn = pl.pallas_call(
        _weights_body,
        out_shape=jax.ShapeDtypeStruct((B, K * K, H, W), jnp.float32),
        grid=(B,),
        in_specs=[pl.BlockSpec((1, 1, H, W), lambda b: (b, 0, 0, 0))],
        out_specs=pl.BlockSpec((1, K * K, H, W), lambda b: (b, 0, 0, 0)),
        compiler_params=pltpu.CompilerParams(
            dimension_semantics=("parallel",),
        ),
        name="bilateral_weights",
    )(depth)

    return pl.pallas_call(
        _pool_body,
        out_shape=jax.ShapeDtypeStruct((B, C, H, W), img.dtype),
        grid=(B, C // _CT),
        in_specs=[
            pl.BlockSpec((1, K * K, H, W), lambda b, c: (b, 0, 0, 0)),
            pl.BlockSpec((1, _CT, H, W), lambda b, c: (b, c, 0, 0)),
        ],
        out_specs=pl.BlockSpec((1, _CT, H, W), lambda b, c: (b, c, 0, 0)),
        compiler_params=pltpu.CompilerParams(
            dimension_semantics=("parallel", "arbitrary"),
            vmem_limit_bytes=56 * 1024 * 1024,
        ),
        name="depth_avg_pool",
    )(wn, img)


# speedup vs baseline: 1.0494x; 1.0494x over previous
import jax
import jax.numpy as jnp
from jax.experimental import pallas as pl
from jax.experimental.pallas import tpu as pltpu

K = 3
ALPHA = 8.3

_CT = 128
_HC = 16
_G = 4
_WHC = 32


def _shift_h(x, o):
    if o == 0:
        return x
    z = jnp.zeros_like(x[..., :1, :])
    if o == 1:
        return jnp.concatenate([x[..., 1:, :], z], axis=-2)
    return jnp.concatenate([z, x[..., :-1, :]], axis=-2)


def _shift_w(x, o):
    if o == 0:
        return x
    z = jnp.zeros_like(x[..., :, :1])
    if o == 1:
        return jnp.concatenate([x[..., :, 1:], z], axis=-1)
    return jnp.concatenate([z, x[..., :, :-1]], axis=-1)


def _body(depth_ref, img_ref, out_ref, wn_ref):
    c_idx = pl.program_id(1)
    H = out_ref.shape[2]
    W = out_ref.shape[3]
    zrow = jnp.zeros((1, W), jnp.float32)

    @pl.when(c_idx == 0)
    def _():
        for h0 in range(0, H, _WHC):
            dc = depth_ref[0, 0, h0:h0 + _WHC, :]
            dh = {}
            for oi in (-1, 0, 1):
                s = h0 + oi
                if s < 0:
                    dh[oi] = jnp.concatenate(
                        [zrow, depth_ref[0, 0, 0:_WHC - 1, :]], axis=0)
                elif s + _WHC > H:
                    dh[oi] = jnp.concatenate(
                        [depth_ref[0, 0, s:H, :], zrow], axis=0)
                else:
                    dh[oi] = depth_ref[0, 0, s:s + _WHC, :]
            ws = []
            for oi in (-1, 0, 1):
                for oj in (-1, 0, 1):
                    dk = _shift_w(dh[oi], oj)
                    ws.append(jnp.exp(-ALPHA * jnp.abs(dc - dk)))
            den = ws[0]
            for w in ws[1:]:
                den = den + w
            inv = 1.0 / den
            k = 0
            for oi in (-1, 0, 1):
                for oj in (-1, 0, 1):
                    wn_ref[k, h0:h0 + _WHC, :] = _shift_w(ws[k] * inv, -oj)
                    k += 1

    def _xh(c, h0):
        xh = {}
        for oi in (-1, 0, 1):
            s = h0 + oi
            if s < 0:
                xh[oi] = jnp.concatenate(
                    [zrow, img_ref[0, c, 0:_HC - 1, :]], axis=0)
            elif s + _HC > H:
                xh[oi] = jnp.concatenate(
                    [img_ref[0, c, s:H, :], zrow], axis=0)
            else:
                xh[oi] = img_ref[0, c, s:s + _HC, :]
        return xh

    for c0 in range(0, _CT, _G):
        for h0 in range(0, H, _HC):
            xhs = [_xh(c0 + g, h0) for g in range(_G)]
            yss = [[None] * K for _ in range(_G)]
            for j_idx in range(K):
                for i_idx, di in enumerate((-1, 0, 1)):
                    w = wn_ref[i_idx * K + j_idx, h0:h0 + _HC, :]
                    for g in range(_G):
                        t = w * xhs[g][di]
                        y = yss[g][j_idx]
                        yss[g][j_idx] = t if y is None else y + t
            for g in range(_G):
                ys = yss[g]
                acc = (pltpu.roll(ys[0], 1, axis=1) + ys[1]
                       + pltpu.roll(ys[2], W - 1, axis=1))
                out_ref[0, c0 + g, h0:h0 + _HC, :] = acc


def kernel(img, depth):
    B, C, H, W = img.shape
    return pl.pallas_call(
        _body,
        out_shape=jax.ShapeDtypeStruct((B, C, H, W), img.dtype),
        grid=(B, C // _CT),
        in_specs=[
            pl.BlockSpec((1, 1, H, W), lambda b, c: (b, 0, 0, 0)),
            pl.BlockSpec((1, _CT, H, W), lambda b, c: (b, c, 0, 0)),
        ],
        out_specs=pl.BlockSpec((1, _CT, H, W), lambda b, c: (b, c, 0, 0)),
        scratch_shapes=[pltpu.VMEM((K * K, H, W), jnp.float32)],
        compiler_params=pltpu.CompilerParams(
            dimension_semantics=("parallel", "arbitrary"),
            vmem_limit_bytes=56 * 1024 * 1024,
        ),
        name="depth_avg_pool",
    )(depth, img)


# speedup vs baseline: 1.0550x; 1.0054x over previous
import jax
import jax.numpy as jnp
from jax.experimental import pallas as pl
from jax.experimental.pallas import tpu as pltpu

K = 3
ALPHA = 8.3

_CT = 128
_HC = 16
_G = 4
_WHC = 32


def _shift_w(x, o):
    if o == 0:
        return x
    z = jnp.zeros_like(x[..., :, :1])
    if o == 1:
        return jnp.concatenate([x[..., :, 1:], z], axis=-1)
    return jnp.concatenate([z, x[..., :, :-1]], axis=-1)


def _compute_wn(depth_ref, wn_ref, buf, h_lo, h_hi):
    H = depth_ref.shape[2]
    W = depth_ref.shape[3]
    zrow = jnp.zeros((1, W), jnp.float32)
    for h0 in range(h_lo, h_hi, _WHC):
        dc = depth_ref[0, 0, h0:h0 + _WHC, :]
        dh = {}
        for oi in (-1, 0, 1):
            s = h0 + oi
            if s < 0:
                dh[oi] = jnp.concatenate(
                    [zrow, depth_ref[0, 0, 0:_WHC - 1, :]], axis=0)
            elif s + _WHC > H:
                dh[oi] = jnp.concatenate(
                    [depth_ref[0, 0, s:H, :], zrow], axis=0)
            else:
                dh[oi] = depth_ref[0, 0, s:s + _WHC, :]
        ws = []
        for oi in (-1, 0, 1):
            for oj in (-1, 0, 1):
                dk = _shift_w(dh[oi], oj)
                ws.append(jnp.exp(-ALPHA * jnp.abs(dc - dk)))
        den = ws[0]
        for w in ws[1:]:
            den = den + w
        inv = 1.0 / den
        k = 0
        for oi in (-1, 0, 1):
            for oj in (-1, 0, 1):
                wn_ref[buf, k, h0:h0 + _WHC, :] = _shift_w(ws[k] * inv, -oj)
                k += 1


def _body(depth_cur_ref, depth_next_ref, img_ref, out_ref, wn_ref):
    b_idx = pl.program_id(0)
    c_idx = pl.program_id(1)
    n_b = pl.num_programs(0)
    H = out_ref.shape[2]
    W = out_ref.shape[3]
    parity = jax.lax.rem(b_idx, 2)
    next_parity = jax.lax.rem(b_idx + 1, 2)

    @pl.when(jnp.logical_and(b_idx == 0, c_idx == 0))
    def _():
        _compute_wn(depth_cur_ref, wn_ref, 0, 0, H)

    @pl.when(jnp.logical_and(b_idx < n_b - 1, c_idx == 0))
    def _():
        _compute_wn(depth_next_ref, wn_ref, next_parity, 0, H // 2)

    @pl.when(jnp.logical_and(b_idx < n_b - 1, c_idx == 1))
    def _():
        _compute_wn(depth_next_ref, wn_ref, next_parity, H // 2, H)

    zrow = jnp.zeros((1, W), jnp.float32)

    def _xh(c, h0):
        xh = {}
        for oi in (-1, 0, 1):
            s = h0 + oi
            if s < 0:
                xh[oi] = jnp.concatenate(
                    [zrow, img_ref[0, c, 0:_HC - 1, :]], axis=0)
            elif s + _HC > H:
                xh[oi] = jnp.concatenate(
                    [img_ref[0, c, s:H, :], zrow], axis=0)
            else:
                xh[oi] = img_ref[0, c, s:s + _HC, :]
        return xh

    for c0 in range(0, _CT, _G):
        for h0 in range(0, H, _HC):
            xhs = [_xh(c0 + g, h0) for g in range(_G)]
            yss = [[None] * K for _ in range(_G)]
            for j_idx in range(K):
                for i_idx, di in enumerate((-1, 0, 1)):
                    w = wn_ref[parity, i_idx * K + j_idx, h0:h0 + _HC, :]
                    for g in range(_G):
                        t = w * xhs[g][di]
                        y = yss[g][j_idx]
                        yss[g][j_idx] = t if y is None else y + t
            for g in range(_G):
                ys = yss[g]
                acc = (pltpu.roll(ys[0], 1, axis=1) + ys[1]
                       + pltpu.roll(ys[2], W - 1, axis=1))
                out_ref[0, c0 + g, h0:h0 + _HC, :] = acc


def kernel(img, depth):
    B, C, H, W = img.shape

    def _next_b(b, c):
        del c
        return jnp.minimum(b + 1, B - 1)

    return pl.pallas_call(
        _body,
        out_shape=jax.ShapeDtypeStruct((B, C, H, W), img.dtype),
        grid=(B, C // _CT),
        in_specs=[
            pl.BlockSpec((1, 1, H, W), lambda b, c: (b, 0, 0, 0)),
            pl.BlockSpec((1, 1, H, W), lambda b, c: (_next_b(b, c), 0, 0, 0)),
            pl.BlockSpec((1, _CT, H, W), lambda b, c: (b, c, 0, 0)),
        ],
        out_specs=pl.BlockSpec((1, _CT, H, W), lambda b, c: (b, c, 0, 0)),
        scratch_shapes=[pltpu.VMEM((2, K * K, H, W), jnp.float32)],
        compiler_params=pltpu.CompilerParams(
            dimension_semantics=("parallel", "arbitrary"),
            vmem_limit_bytes=56 * 1024 * 1024,
        ),
        name="depth_avg_pool",
    )(depth, depth, img)


# speedup vs baseline: 1.0566x; 1.0015x over previous
import jax
import jax.numpy as jnp
from jax.experimental import pallas as pl
from jax.experimental.pallas import tpu as pltpu

K = 3
ALPHA = 8.3

_CT = 128
_HC = 16
_G = 4


def _shift_h(x, o):
    if o == 0:
        return x
    z = jnp.zeros_like(x[..., :1, :])
    if o == 1:
        return jnp.concatenate([x[..., 1:, :], z], axis=-2)
    return jnp.concatenate([z, x[..., :-1, :]], axis=-2)


def _shift_w(x, o):
    if o == 0:
        return x
    z = jnp.zeros_like(x[..., :, :1])
    if o == 1:
        return jnp.concatenate([x[..., :, 1:], z], axis=-1)
    return jnp.concatenate([z, x[..., :, :-1]], axis=-1)


def _body(depth_ref, img_ref, out_ref, wn_ref):
    c_idx = pl.program_id(1)
    H = out_ref.shape[2]
    W = out_ref.shape[3]

    @pl.when(c_idx == 0)
    def _():
        d = depth_ref[0, 0]
        ws = []
        for oi in (-1, 0, 1):
            dh = _shift_h(d, oi)
            for oj in (-1, 0, 1):
                dk = _shift_w(dh, oj)
                ws.append(jnp.exp(-ALPHA * jnp.abs(d - dk)))
        den = ws[0]
        for w in ws[1:]:
            den = den + w
        inv = 1.0 / den
        k = 0
        for oi in (-1, 0, 1):
            for oj in (-1, 0, 1):
                wn_ref[k] = _shift_w(ws[k] * inv, -oj)
                k += 1

    zrow = jnp.zeros((1, W), jnp.float32)

    def _xh(c, h0):
        xh = {}
        for oi in (-1, 0, 1):
            s = h0 + oi
            if s < 0:
                xh[oi] = jnp.concatenate(
                    [zrow, img_ref[0, c, 0:_HC - 1, :]], axis=0)
            elif s + _HC > H:
                xh[oi] = jnp.concatenate(
                    [img_ref[0, c, s:H, :], zrow], axis=0)
            else:
                xh[oi] = img_ref[0, c, s:s + _HC, :]
        return xh

    for c0 in range(0, _CT, _G):
        for h0 in range(0, H, _HC):
            xhs = [_xh(c0 + g, h0) for g in range(_G)]
            yss = [[None] * K for _ in range(_G)]
            for j_idx in range(K):
                for i_idx, di in enumerate((-1, 0, 1)):
                    w = wn_ref[i_idx * K + j_idx, h0:h0 + _HC, :]
                    for g in range(_G):
                        t = w * xhs[g][di]
                        y = yss[g][j_idx]
                        yss[g][j_idx] = t if y is None else y + t
            for g in range(_G):
                ys = yss[g]
                acc = (pltpu.roll(ys[0], 1, axis=1) + ys[1]
                       + pltpu.roll(ys[2], W - 1, axis=1))
                out_ref[0, c0 + g, h0:h0 + _HC, :] = acc


def kernel(img, depth):
    B, C, H, W = img.shape
    return pl.pallas_call(
        _body,
        out_shape=jax.ShapeDtypeStruct((B, C, H, W), img.dtype),
        grid=(B, C // _CT),
        in_specs=[
            pl.BlockSpec((1, 1, H, W), lambda b, c: (b, 0, 0, 0)),
            pl.BlockSpec((1, _CT, H, W), lambda b, c: (b, c, 0, 0)),
        ],
        out_specs=pl.BlockSpec((1, _CT, H, W), lambda b, c: (b, c, 0, 0)),
        scratch_shapes=[pltpu.VMEM((K * K, H, W), jnp.float32)],
        compiler_params=pltpu.CompilerParams(
            dimension_semantics=("parallel", "arbitrary"),
            vmem_limit_bytes=56 * 1024 * 1024,
        ),
        name="depth_avg_pool",
    )(depth, img)
